# Initial kernel scaffold; baseline (speedup 1.0000x reference)
import jax
import jax.numpy as jnp
from jax.experimental import pallas as pl


def kernel(y_pred, box_tensor):
    raise NotImplementedError("write your pallas kernel here")


# speedup vs baseline: 40.4019x; 40.4019x over previous
import jax
import jax.numpy as jnp
from jax.experimental import pallas as pl
from jax.experimental.pallas import tpu as pltpu

_N_BOXES = 32766
_N_PAD = 32768
_R = 256
_C = 128
_N_CLASSES = 21
_CONF_THRESH = 0.01
_IOU_THRESH = 0.45
_TOP_K = 100
_NEG = -1e9


def _nms_body(y_ref, box_ref, out_ref, x1s, y1s, x2s, y2s, areas, clss, works):
    best = y_ref[0, 0]
    cid = jnp.zeros((_R, _C), jnp.int32)
    for k in range(1, _N_CLASSES):
        s = y_ref[0, k]
        gt = s > best
        cid = jnp.where(gt, k, cid)
        best = jnp.maximum(best, s)
    conf = best

    bcx = box_ref[0]
    bcy = box_ref[1]
    bw = box_ref[2]
    bh = box_ref[3]
    cx = y_ref[0, _N_CLASSES + 0] * 0.1 * bw + bcx
    cy = y_ref[0, _N_CLASSES + 1] * 0.1 * bh + bcy
    w = jnp.exp(y_ref[0, _N_CLASSES + 2] * 0.2) * bw
    h = jnp.exp(y_ref[0, _N_CLASSES + 3] * 0.2) * bh
    x1 = jnp.clip(cx - 0.5 * w, 0.0, 1.0)
    y1 = jnp.clip(cy - 0.5 * h, 0.0, 1.0)
    x2 = jnp.clip(cx + 0.5 * w, 0.0, 1.0)
    y2 = jnp.clip(cy + 0.5 * h, 0.0, 1.0)

    idx = (jax.lax.broadcasted_iota(jnp.int32, (_R, _C), 0) * _C
           + jax.lax.broadcasted_iota(jnp.int32, (_R, _C), 1))
    valid = (cid != 0) & (conf > _CONF_THRESH) & (idx < _N_BOXES)

    x1s[...] = x1
    y1s[...] = y1
    x2s[...] = x2
    y2s[...] = y2
    areas[...] = (x2 - x1) * (y2 - y1)
    clss[...] = cid.astype(jnp.float32)
    works[...] = jnp.where(valid, conf, _NEG)

    def body(t, carry):
        work = works[...]
        m = jnp.max(work)
        i_sel = jnp.min(jnp.where(work == m, idx, jnp.int32(_N_PAD)))
        onehot = idx == i_sel
        z = jnp.float32(0.0)
        x1v = x1s[...]
        y1v = y1s[...]
        x2v = x2s[...]
        y2v = y2s[...]
        av = areas[...]
        bx1 = jnp.sum(jnp.where(onehot, x1v, z))
        by1 = jnp.sum(jnp.where(onehot, y1v, z))
        bx2 = jnp.sum(jnp.where(onehot, x2v, z))
        by2 = jnp.sum(jnp.where(onehot, y2v, z))
        ba = jnp.sum(jnp.where(onehot, av, z))
        bc = jnp.sum(jnp.where(onehot, clss[...], z))
        keep = m > -1e8

        lane6 = jax.lax.broadcasted_iota(jnp.int32, (1, 6), 1)
        row = jnp.where(lane6 == 0, bc,
              jnp.where(lane6 == 1, m,
              jnp.where(lane6 == 2, bx1,
              jnp.where(lane6 == 3, by1,
              jnp.where(lane6 == 4, bx2, by2)))))
        row = jnp.where(keep, row, 0.0)
        out_ref[0, pl.ds(t, 1), :] = row

        ix1 = jnp.maximum(x1v, bx1)
        iy1 = jnp.maximum(y1v, by1)
        ix2 = jnp.minimum(x2v, bx2)
        iy2 = jnp.minimum(y2v, by2)
        inter = jnp.maximum(ix2 - ix1, 0.0) * jnp.maximum(iy2 - iy1, 0.0)
        union = av + ba - inter
        iou = inter / jnp.maximum(union, 1e-8)
        suppress = (iou > _IOU_THRESH) & keep
        works[...] = jnp.where(suppress | onehot, _NEG, work)
        return carry

    jax.lax.fori_loop(0, _TOP_K, body, 0)


def kernel(y_pred, box_tensor):
    b = y_pred.shape[0]
    nc = _N_CLASSES + 4
    y_pad = jnp.pad(y_pred, ((0, 0), (0, _N_PAD - _N_BOXES), (0, 0)))
    y4 = y_pad.transpose(0, 2, 1).reshape(b, nc, _R, _C)
    box_pad = jnp.pad(box_tensor, ((0, _N_PAD - _N_BOXES), (0, 0)))
    box3 = box_pad.T.reshape(4, _R, _C)

    f32 = jnp.float32
    return pl.pallas_call(
        _nms_body,
        grid=(b,),
        in_specs=[
            pl.BlockSpec((1, nc, _R, _C), lambda i: (i, 0, 0, 0)),
            pl.BlockSpec((4, _R, _C), lambda i: (0, 0, 0)),
        ],
        out_specs=pl.BlockSpec((1, _TOP_K, 6), lambda i: (i, 0, 0)),
        out_shape=jax.ShapeDtypeStruct((b, _TOP_K, 6), f32),
        scratch_shapes=[pltpu.VMEM((_R, _C), f32) for _ in range(7)],
    )(y4, box3)


# speedup vs baseline: 44.6126x; 1.1042x over previous
import jax
import jax.numpy as jnp
from jax.experimental import pallas as pl
from jax.experimental.pallas import tpu as pltpu

_N_BOXES = 32766
_N_PAD = 32768
_R = 256
_C = 128
_N_CLASSES = 21
_CONF_THRESH = 0.01
_IOU_THRESH = 0.45
_TOP_K = 100
_NEG = -1e9


def _nms_body(y_ref, box_ref, out_ref, x1s, y1s, x2s, y2s, areas, clss, works):
    best = y_ref[0, 0]
    cid = jnp.zeros((_R, _C), jnp.int32)
    for k in range(1, _N_CLASSES):
        s = y_ref[0, k]
        gt = s > best
        cid = jnp.where(gt, k, cid)
        best = jnp.maximum(best, s)
    conf = best

    bcx = box_ref[0]
    bcy = box_ref[1]
    bw = box_ref[2]
    bh = box_ref[3]
    cx = y_ref[0, _N_CLASSES + 0] * 0.1 * bw + bcx
    cy = y_ref[0, _N_CLASSES + 1] * 0.1 * bh + bcy
    w = jnp.exp(y_ref[0, _N_CLASSES + 2] * 0.2) * bw
    h = jnp.exp(y_ref[0, _N_CLASSES + 3] * 0.2) * bh
    x1 = jnp.clip(cx - 0.5 * w, 0.0, 1.0)
    y1 = jnp.clip(cy - 0.5 * h, 0.0, 1.0)
    x2 = jnp.clip(cx + 0.5 * w, 0.0, 1.0)
    y2 = jnp.clip(cy + 0.5 * h, 0.0, 1.0)

    idx = (jax.lax.broadcasted_iota(jnp.int32, (_R, _C), 0) * _C
           + jax.lax.broadcasted_iota(jnp.int32, (_R, _C), 1))
    valid = (cid != 0) & (conf > _CONF_THRESH) & (idx < _N_BOXES)

    x1s[...] = x1
    y1s[...] = y1
    x2s[...] = x2
    y2s[...] = y2
    areas[...] = (x2 - x1) * (y2 - y1)
    clss[...] = cid.astype(jnp.float32)
    works[...] = jnp.where(valid, conf, _NEG)

    lane = jax.lax.broadcasted_iota(jnp.int32, (1, _C), 1)

    def body(t, carry):
        work = works[...]
        m = jnp.max(work)
        i_sel = jnp.min(jnp.where(work == m, idx, jnp.int32(_N_PAD)))
        r = i_sel // _C
        c = i_sel % _C
        oh = lane == c
        z = jnp.float32(0.0)
        bx1 = jnp.sum(jnp.where(oh, x1s[pl.ds(r, 1), :], z))
        by1 = jnp.sum(jnp.where(oh, y1s[pl.ds(r, 1), :], z))
        bx2 = jnp.sum(jnp.where(oh, x2s[pl.ds(r, 1), :], z))
        by2 = jnp.sum(jnp.where(oh, y2s[pl.ds(r, 1), :], z))
        ba = jnp.sum(jnp.where(oh, areas[pl.ds(r, 1), :], z))
        bc = jnp.sum(jnp.where(oh, clss[pl.ds(r, 1), :], z))
        keep = m > -1e8

        lane6 = jax.lax.broadcasted_iota(jnp.int32, (1, 6), 1)
        row = jnp.where(lane6 == 0, bc,
              jnp.where(lane6 == 1, m,
              jnp.where(lane6 == 2, bx1,
              jnp.where(lane6 == 3, by1,
              jnp.where(lane6 == 4, bx2, by2)))))
        row = jnp.where(keep, row, 0.0)
        out_ref[0, pl.ds(t, 1), :] = row

        ix1 = jnp.maximum(x1s[...], bx1)
        iy1 = jnp.maximum(y1s[...], by1)
        ix2 = jnp.minimum(x2s[...], bx2)
        iy2 = jnp.minimum(y2s[...], by2)
        inter = jnp.maximum(ix2 - ix1, 0.0) * jnp.maximum(iy2 - iy1, 0.0)
        union = areas[...] + ba - inter
        iou = inter / jnp.maximum(union, 1e-8)
        suppress = (iou > _IOU_THRESH) & keep
        works[...] = jnp.where(suppress | (idx == i_sel), _NEG, work)
        return carry

    jax.lax.fori_loop(0, _TOP_K, body, 0)


def kernel(y_pred, box_tensor):
    b = y_pred.shape[0]
    nc = _N_CLASSES + 4
    y_pad = jnp.pad(y_pred, ((0, 0), (0, _N_PAD - _N_BOXES), (0, 0)))
    y4 = y_pad.transpose(0, 2, 1).reshape(b, nc, _R, _C)
    box_pad = jnp.pad(box_tensor, ((0, _N_PAD - _N_BOXES), (0, 0)))
    box3 = box_pad.T.reshape(4, _R, _C)

    f32 = jnp.float32
    return pl.pallas_call(
        _nms_body,
        grid=(b,),
        in_specs=[
            pl.BlockSpec((1, nc, _R, _C), lambda i: (i, 0, 0, 0)),
            pl.BlockSpec((4, _R, _C), lambda i: (0, 0, 0)),
        ],
        out_specs=pl.BlockSpec((1, _TOP_K, 6), lambda i: (i, 0, 0)),
        out_shape=jax.ShapeDtypeStruct((b, _TOP_K, 6), f32),
        scratch_shapes=[pltpu.VMEM((_R, _C), f32) for _ in range(7)],
    )(y4, box3)
